# Initial kernel scaffold; baseline (speedup 1.0000x reference)
import functools

import jax
import jax.numpy as jnp
from jax import lax
from jax.experimental import pallas as pl
from jax.experimental.pallas import tpu as pltpu
from jax.experimental.pallas import tpu_sc as plsc

ROWS = 64
COLS = 32768
L = 16
NC = 2
NS = 16
NW = NC * NS
ROWS_PER_W = ROWS // NW
UNROLL = 8


def _top2_body(x_hbm, out_hbm, row_v, res_v):
    c = lax.axis_index("c")
    s = lax.axis_index("s")
    wid = s * NC + c
    lanes = lax.iota(jnp.int32, L)
    neg_inf = jnp.full((L,), -jnp.inf, jnp.float32)

    for r in range(ROWS_PER_W):
        row = wid * ROWS_PER_W + r
        pltpu.sync_copy(x_hbm.at[row], row_v)

        def body(i, carry):
            m1, m2 = carry
            base = i * (L * UNROLL)
            for u in range(UNROLL):
                v = row_v[pl.ds(base + u * L, L)]
                m2 = jnp.maximum(m2, jnp.minimum(m1, v))
                m1 = jnp.maximum(m1, v)
            return (m1, m2)

        m1, m2 = lax.fori_loop(0, COLS // (L * UNROLL), body,
                               (neg_inf, neg_inf))

        t1 = jnp.max(m1)
        ffs = plsc.all_reduce_ffs(m1 == t1)
        m1_excl = jnp.where(lanes == ffs, neg_inf, m1)
        t2 = jnp.maximum(jnp.max(m1_excl), jnp.max(m2))

        res_v[...] = jnp.where(lanes == 0, t1,
                               jnp.where(lanes == 1, t2, 0.0))
        pltpu.sync_copy(res_v, out_hbm.at[row])


@jax.jit
def _top2_sc(x):
    mesh = plsc.VectorSubcoreMesh(core_axis_name="c", subcore_axis_name="s")
    call = functools.partial(
        pl.kernel,
        mesh=mesh,
        out_type=jax.ShapeDtypeStruct((ROWS, L), jnp.float32),
        scratch_types=[
            pltpu.VMEM((COLS,), jnp.float32),
            pltpu.VMEM((L,), jnp.float32),
        ],
    )(_top2_body)
    return call(x)


def kernel(x):
    vals = _top2_sc(x)
    return vals[0:0, 0:1]


# speedup vs baseline: 1.0366x; 1.0366x over previous
import functools

import jax
import jax.numpy as jnp
from jax import lax
from jax.experimental import pallas as pl
from jax.experimental.pallas import tpu as pltpu
from jax.experimental.pallas import tpu_sc as plsc

ROWS = 64
COLS = 32768
L = 16
NC = 2
NS = 16
NW = NC * NS
ROWS_PER_W = ROWS // NW
UNROLL = 8


def _top2_body(x_hbm, out_hbm, row_v, res_v):
    c = lax.axis_index("c")
    s = lax.axis_index("s")
    wid = s * NC + c
    lanes = lax.iota(jnp.int32, L)
    neg_inf = jnp.full((L,), -jnp.inf, jnp.float32)

    for r in range(ROWS_PER_W):
        row = wid * ROWS_PER_W + r
        pltpu.sync_copy(x_hbm.at[row], row_v)

        def body(i, carry):
            m1, m2 = carry
            base = i * (L * UNROLL)
            for u in range(UNROLL):
                v = row_v[pl.ds(base + u * L, L)]
                m2 = jnp.maximum(m2, jnp.minimum(m1, v))
                m1 = jnp.maximum(m1, v)
            return (m1, m2)

        m1, m2 = lax.fori_loop(0, COLS // (L * UNROLL), body,
                               (neg_inf, neg_inf))

        t1 = jnp.max(m1)
        ffs = plsc.all_reduce_ffs(m1 == t1)
        m1_excl = jnp.where(lanes == ffs, neg_inf, m1)
        t2 = jnp.maximum(jnp.max(m1_excl), jnp.max(m2))

        res_v[...] = jnp.where(lanes == 0, t1,
                               jnp.where(lanes == 1, t2, 0.0))
        pltpu.sync_copy(res_v, out_hbm.at[row])


@jax.jit
def _top2_sc(x):
    mesh = plsc.VectorSubcoreMesh(core_axis_name="c", subcore_axis_name="s")
    call = functools.partial(
        pl.kernel,
        mesh=mesh,
        out_type=jax.ShapeDtypeStruct((ROWS, L), jnp.float32),
        scratch_types=[
            pltpu.VMEM((COLS,), jnp.float32),
            pltpu.VMEM((L,), jnp.float32),
        ],
        compiler_params=pltpu.CompilerParams(needs_layout_passes=False),
    )(_top2_body)
    return call(x)


def kernel(x):
    vals = _top2_sc(x)
    return vals[0:0, 0:1]
